# baseline (speedup 1.0000x reference)
import functools

import jax
import jax.numpy as jnp
import numpy as np
from jax.experimental import pallas as pl
from jax.experimental.pallas import tpu as pltpu

_N_HID = 32
_NB_ITER = 1
_K = 32
_B, _N, _M = 2, 4096, 4096


_BI = 256


def _ktile(f0b, f1, p1b, p2, rec_eps):
    f0n = f0b * jax.lax.rsqrt(jnp.sum(f0b * f0b, -1, keepdims=True) + 1e-8)
    f1n = f1 * jax.lax.rsqrt(jnp.sum(f1 * f1, -1, keepdims=True) + 1e-8)
    c = 1.0 - jax.lax.dot_general(
        f0n, f1n, (((1,), (1,)), ((), ())), preferred_element_type=jnp.float32)
    d2 = jnp.sum(p1b * p1b, -1, keepdims=True) + jnp.sum(p2 * p2, -1)[None, :]
    d2 = d2 - 2.0 * jax.lax.dot_general(
        p1b, p2, (((1,), (1,)), ((), ())), preferred_element_type=jnp.float32)
    return jnp.exp(-c * rec_eps) * (d2 < 100.0).astype(jnp.float32)


def _colsum_body(scal, f0, f1, p1, p2, out):
    i = pl.program_id(1)
    kt = _ktile(f0[0], f1[0], p1[0], p2[0], scal[0])
    part = jnp.sum(kt, axis=0, keepdims=True)

    @pl.when(i == 0)
    def _():
        out[0] = part

    @pl.when(i != 0)
    def _():
        out[0] += part


def _rows_body(scal, f0, f1, p1, p2, colsum, out):
    rec_eps, power, inv_n, inv_m = scal[0], scal[1], scal[2], scal[3]
    kt = _ktile(f0[0], f1[0], p1[0], p2[0], rec_eps)
    bb = jnp.exp(power * jnp.log(inv_m / (colsum[0, 0] * inv_n + 1e-8)))
    kb_t = kt * bb[None, :]
    kb = jnp.sum(kb_t, -1, keepdims=True)
    a = jnp.exp(power * jnp.log(inv_n / (kb + 1e-8)))
    s = jnp.dot(kb_t, p2[0], preferred_element_type=jnp.float32)
    out[0] = (a * s) / (a * kb + 1e-8) - p1[0]


def _sinkhorn_flow(f0, f1, pc1, pc2, eps, gam):
    power = gam / (gam + eps)
    scal = jnp.stack([1.0 / eps, power, 1.0 / _N, 1.0 / _M]).astype(jnp.float32)
    n_i = _N // _BI
    colsum = pl.pallas_call(
        _colsum_body,
        grid=(_B, n_i),
        in_specs=[
            pl.BlockSpec(memory_space=pltpu.SMEM),
            pl.BlockSpec((1, _BI, f0.shape[-1]), lambda b, i: (b, i, 0)),
            pl.BlockSpec((1, _M, f1.shape[-1]), lambda b, i: (b, 0, 0)),
            pl.BlockSpec((1, _BI, 3), lambda b, i: (b, i, 0)),
            pl.BlockSpec((1, _M, 3), lambda b, i: (b, 0, 0)),
        ],
        out_specs=pl.BlockSpec((1, 1, _M), lambda b, i: (b, 0, 0)),
        out_shape=jax.ShapeDtypeStruct((_B, 1, _M), jnp.float32),
    )(scal, f0, f1, pc1, pc2)
    flow = pl.pallas_call(
        _rows_body,
        grid=(_B, n_i),
        in_specs=[
            pl.BlockSpec(memory_space=pltpu.SMEM),
            pl.BlockSpec((1, _BI, f0.shape[-1]), lambda b, i: (b, i, 0)),
            pl.BlockSpec((1, _M, f1.shape[-1]), lambda b, i: (b, 0, 0)),
            pl.BlockSpec((1, _BI, 3), lambda b, i: (b, i, 0)),
            pl.BlockSpec((1, _M, 3), lambda b, i: (b, 0, 0)),
            pl.BlockSpec((1, 1, _M), lambda b, i: (b, 0, 0)),
        ],
        out_specs=pl.BlockSpec((1, _BI, 3), lambda b, i: (b, i, 0)),
        out_shape=jax.ShapeDtypeStruct((_B, _N, 3), jnp.float32),
    )(scal, f0, f1, pc1, pc2, colsum)
    return flow


_RB = 128
_SEG = 512


def _knn_body(h_rows, g_cols, out, d_ref, acc_ref, pidx_ref):
    t = pl.program_id(2)
    n = g_cols.shape[1]
    nseg = n // _SEG
    big = jnp.int32(2 ** 30)

    @pl.when(t == 0)
    def _():
        hr = h_rows[0]

        def init(s, _):
            ga = g_cols[0, pl.ds(s * _SEG, _SEG)]
            s3 = ga[:, 0:1] * hr[0:1, :]
            s3 = s3 + ga[:, 1:2] * hr[1:2, :]
            s3 = s3 + ga[:, 2:3] * hr[2:3, :]
            d_ref[pl.ds(s * _SEG, _SEG), :] = (
                jnp.broadcast_to(ga[:, 3:4], (_SEG, _RB)) + s3)
            return 0

        jax.lax.fori_loop(0, nseg, init, 0, unroll=2)
        pidx_ref[...] = jnp.full((1, _RB), -1, jnp.int32)

    pidx = pidx_ref[...]

    def scan(s, carry):
        m, idx = carry
        seg = d_ref[pl.ds(s * _SEG, _SEG), :]
        jidx = jax.lax.broadcasted_iota(jnp.int32, (_SEG, _RB), 0) + s * _SEG
        seg = jnp.where(jidx == pidx, jnp.float32(1e30), seg)
        d_ref[pl.ds(s * _SEG, _SEG), :] = seg
        smin = jnp.min(seg, 0, keepdims=True)
        scand = jnp.min(jnp.where(seg == smin, jidx, big), 0, keepdims=True)
        better = smin < m
        return (jnp.where(better, smin, m), jnp.where(better, scand, idx))

    m0 = jnp.full((1, _RB), 1e30, jnp.float32)
    i0 = jnp.full((1, _RB), big, jnp.int32)
    _, idx = jax.lax.fori_loop(0, nseg, scan, (m0, i0), unroll=2)
    pidx_ref[...] = idx
    acc_ref[pl.ds(t, 1)] = idx

    @pl.when(t == _K - 1)
    def _():
        out[0] = acc_ref[...]


def _build_graph(pcloud, k):
    b, n, _ = pcloud.shape
    p2sum = jnp.sum(pcloud * pcloud, -1, keepdims=True)
    pr = jax.lax.reduce_precision(pcloud, 8, 7)
    g = jnp.concatenate([pr, p2sum], -1)
    h = -2.0 * jnp.swapaxes(pr, 1, 2)
    nbr_t = pl.pallas_call(
        _knn_body,
        grid=(b, n // _RB, _K),
        in_specs=[
            pl.BlockSpec((1, 3, _RB), lambda bb, i, t: (bb, 0, i)),
            pl.BlockSpec((1, n, 4), lambda bb, i, t: (bb, 0, 0)),
        ],
        out_specs=pl.BlockSpec((1, _K, _RB), lambda bb, i, t: (bb, 0, i)),
        out_shape=jax.ShapeDtypeStruct((b, _K, n), jnp.int32),
        scratch_shapes=[pltpu.VMEM((n, _RB), jnp.float32),
                        pltpu.VMEM((_K, _RB), jnp.int32),
                        pltpu.VMEM((1, _RB), jnp.int32)],
    )(h, g)
    neighbors = jnp.swapaxes(nbr_t, 1, 2)
    edges = (neighbors + jnp.arange(b)[:, None, None] * n).reshape(-1)
    return edges


def _gnorm(x, w, b, num_groups=8, eps=1e-5):
    sh = x.shape
    c = sh[-1]
    xg = x.reshape(sh[:-1] + (num_groups, c // num_groups))
    mean = jnp.mean(xg, -1, keepdims=True)
    var = jnp.var(xg, -1, keepdims=True)
    xg = (xg - mean) / jnp.sqrt(var + eps)
    return xg.reshape(sh) * w + b


def _set_conv(p, signal, pts_flat, edges):
    b, n, c = signal.shape
    k = _K
    cout = p['W1'].shape[0]
    w1a = p['W1'][:, :c]
    w1b = p['W1'][:, c:]
    sig = signal.reshape(b * n, c)
    pw = pts_flat @ w1b.T
    u = sig @ w1a.T + pw
    v = p['b1'][None, :] - pw
    h = u[edges].reshape(b * n, k, cout) + v[:, None, :]
    h = _gnorm(h, p['gn1_w'], p['gn1_b'])
    h = jax.nn.leaky_relu(h, 0.1)
    h = jnp.max(h, axis=1)
    h = h @ p['W2'].T + p['b2']
    h = _gnorm(h, p['gn2_w'], p['gn2_b'])
    h = jax.nn.leaky_relu(h, 0.1)
    h = h @ p['W3'].T + p['b3']
    h = _gnorm(h, p['gn3_w'], p['gn3_b'])
    h = jax.nn.leaky_relu(h, 0.1)
    return h.reshape(b, n, cout)


def kernel(pc1, pc2, params):
    p1f = pc1.reshape(_B * _N, 3)
    p2f = pc2.reshape(_B * _M, 3)
    e1 = _build_graph(pc1, _K)
    f0 = _set_conv(params['feat_conv1'], pc1, p1f, e1)
    f0 = _set_conv(params['feat_conv2'], f0, p1f, e1)
    f0 = _set_conv(params['feat_conv3'], f0, p1f, e1)
    e2 = _build_graph(pc2, _K)
    f1 = _set_conv(params['feat_conv1'], pc2, p2f, e2)
    f1 = _set_conv(params['feat_conv2'], f1, p2f, e2)
    f1 = _set_conv(params['feat_conv3'], f1, p2f, e2)
    eps = jnp.exp(params['epsilon'])[0] + 0.03
    gam = jnp.exp(params['gamma'])[0]
    ot_flow = _sinkhorn_flow(f0, f1, pc1, pc2, eps, gam)
    x = _set_conv(params['ref_conv1'], ot_flow, p1f, e1)
    x = _set_conv(params['ref_conv2'], x, p1f, e1)
    x = _set_conv(params['ref_conv3'], x, p1f, e1)
    x = x.reshape(_B * _N, -1) @ params['fc_W'].T + params['fc_b']
    return ot_flow + x.reshape(_B, _N, 3)


# speedup vs baseline: 1.4996x; 1.4996x over previous
import functools

import jax
import jax.numpy as jnp
import numpy as np
from jax.experimental import pallas as pl
from jax.experimental.pallas import tpu as pltpu
from jax.experimental.pallas import tpu_sc as plsc

_N_HID = 32
_NB_ITER = 1
_K = 32
_B, _N, _M = 2, 4096, 4096


_BI = 256


def _ktile(f0b, f1, p1b, p2, rec_eps):
    f0n = f0b * jax.lax.rsqrt(jnp.sum(f0b * f0b, -1, keepdims=True) + 1e-8)
    f1n = f1 * jax.lax.rsqrt(jnp.sum(f1 * f1, -1, keepdims=True) + 1e-8)
    c = 1.0 - jax.lax.dot_general(
        f0n, f1n, (((1,), (1,)), ((), ())), preferred_element_type=jnp.float32)
    d2 = jnp.sum(p1b * p1b, -1, keepdims=True) + jnp.sum(p2 * p2, -1)[None, :]
    d2 = d2 - 2.0 * jax.lax.dot_general(
        p1b, p2, (((1,), (1,)), ((), ())), preferred_element_type=jnp.float32)
    return jnp.exp(-c * rec_eps) * (d2 < 100.0).astype(jnp.float32)


def _colsum_body(scal, f0, f1, p1, p2, out):
    i = pl.program_id(1)
    kt = _ktile(f0[0], f1[0], p1[0], p2[0], scal[0])
    part = jnp.sum(kt, axis=0, keepdims=True)

    @pl.when(i == 0)
    def _():
        out[0] = part

    @pl.when(i != 0)
    def _():
        out[0] += part


def _rows_body(scal, f0, f1, p1, p2, colsum, out):
    rec_eps, power, inv_n, inv_m = scal[0], scal[1], scal[2], scal[3]
    kt = _ktile(f0[0], f1[0], p1[0], p2[0], rec_eps)
    bb = jnp.exp(power * jnp.log(inv_m / (colsum[0, 0] * inv_n + 1e-8)))
    kb_t = kt * bb[None, :]
    kb = jnp.sum(kb_t, -1, keepdims=True)
    a = jnp.exp(power * jnp.log(inv_n / (kb + 1e-8)))
    s = jnp.dot(kb_t, p2[0], preferred_element_type=jnp.float32)
    out[0] = (a * s) / (a * kb + 1e-8) - p1[0]


def _sinkhorn_flow(f0, f1, pc1, pc2, eps, gam):
    power = gam / (gam + eps)
    scal = jnp.stack([1.0 / eps, power, 1.0 / _N, 1.0 / _M]).astype(jnp.float32)
    n_i = _N // _BI
    colsum = pl.pallas_call(
        _colsum_body,
        grid=(_B, n_i),
        in_specs=[
            pl.BlockSpec(memory_space=pltpu.SMEM),
            pl.BlockSpec((1, _BI, f0.shape[-1]), lambda b, i: (b, i, 0)),
            pl.BlockSpec((1, _M, f1.shape[-1]), lambda b, i: (b, 0, 0)),
            pl.BlockSpec((1, _BI, 3), lambda b, i: (b, i, 0)),
            pl.BlockSpec((1, _M, 3), lambda b, i: (b, 0, 0)),
        ],
        out_specs=pl.BlockSpec((1, 1, _M), lambda b, i: (b, 0, 0)),
        out_shape=jax.ShapeDtypeStruct((_B, 1, _M), jnp.float32),
    )(scal, f0, f1, pc1, pc2)
    flow = pl.pallas_call(
        _rows_body,
        grid=(_B, n_i),
        in_specs=[
            pl.BlockSpec(memory_space=pltpu.SMEM),
            pl.BlockSpec((1, _BI, f0.shape[-1]), lambda b, i: (b, i, 0)),
            pl.BlockSpec((1, _M, f1.shape[-1]), lambda b, i: (b, 0, 0)),
            pl.BlockSpec((1, _BI, 3), lambda b, i: (b, i, 0)),
            pl.BlockSpec((1, _M, 3), lambda b, i: (b, 0, 0)),
            pl.BlockSpec((1, 1, _M), lambda b, i: (b, 0, 0)),
        ],
        out_specs=pl.BlockSpec((1, _BI, 3), lambda b, i: (b, i, 0)),
        out_shape=jax.ShapeDtypeStruct((_B, _N, 3), jnp.float32),
    )(scal, f0, f1, pc1, pc2, colsum)
    return flow


_RB = 128
_SEG = 512


def _knn_body(h_rows, g_cols, out, d_ref, acc_ref, pidx_ref):
    t = pl.program_id(2)
    n = g_cols.shape[1]
    nseg = n // _SEG
    big = jnp.int32(2 ** 30)

    @pl.when(t == 0)
    def _():
        hr = h_rows[0]

        def init(s, _):
            ga = g_cols[0, pl.ds(s * _SEG, _SEG)]
            s3 = ga[:, 0:1] * hr[0:1, :]
            s3 = s3 + ga[:, 1:2] * hr[1:2, :]
            s3 = s3 + ga[:, 2:3] * hr[2:3, :]
            d_ref[pl.ds(s * _SEG, _SEG), :] = (
                jnp.broadcast_to(ga[:, 3:4], (_SEG, _RB)) + s3)
            return 0

        jax.lax.fori_loop(0, nseg, init, 0, unroll=2)
        pidx_ref[...] = jnp.full((1, _RB), -1, jnp.int32)

    pidx = pidx_ref[...]

    def scan(s, carry):
        m, idx = carry
        seg = d_ref[pl.ds(s * _SEG, _SEG), :]
        jidx = jax.lax.broadcasted_iota(jnp.int32, (_SEG, _RB), 0) + s * _SEG
        seg = jnp.where(jidx == pidx, jnp.float32(1e30), seg)
        d_ref[pl.ds(s * _SEG, _SEG), :] = seg
        smin = jnp.min(seg, 0, keepdims=True)
        scand = jnp.min(jnp.where(seg == smin, jidx, big), 0, keepdims=True)
        better = smin < m
        return (jnp.where(better, smin, m), jnp.where(better, scand, idx))

    m0 = jnp.full((1, _RB), 1e30, jnp.float32)
    i0 = jnp.full((1, _RB), big, jnp.int32)
    _, idx = jax.lax.fori_loop(0, nseg, scan, (m0, i0), unroll=2)
    pidx_ref[...] = idx
    acc_ref[pl.ds(t, 1)] = idx

    @pl.when(t == _K - 1)
    def _():
        out[0] = acc_ref[...]


def _build_graph(pcloud, k):
    b, n, _ = pcloud.shape
    p2sum = jnp.sum(pcloud * pcloud, -1, keepdims=True)
    pr = jax.lax.reduce_precision(pcloud, 8, 7)
    g = jnp.concatenate([pr, p2sum], -1)
    h = -2.0 * jnp.swapaxes(pr, 1, 2)
    nbr_t = pl.pallas_call(
        _knn_body,
        grid=(b, n // _RB, _K),
        in_specs=[
            pl.BlockSpec((1, 3, _RB), lambda bb, i, t: (bb, 0, i)),
            pl.BlockSpec((1, n, 4), lambda bb, i, t: (bb, 0, 0)),
        ],
        out_specs=pl.BlockSpec((1, _K, _RB), lambda bb, i, t: (bb, 0, i)),
        out_shape=jax.ShapeDtypeStruct((b, _K, n), jnp.int32),
        scratch_shapes=[pltpu.VMEM((n, _RB), jnp.float32),
                        pltpu.VMEM((_K, _RB), jnp.int32),
                        pltpu.VMEM((1, _RB), jnp.int32)],
    )(h, g)
    neighbors = jnp.swapaxes(nbr_t, 1, 2)
    edges = (neighbors + jnp.arange(b)[:, None, None] * n).reshape(-1)
    return edges


def _sc_gather(table, idx):
    v, d0 = table.shape
    if d0 % 128:
        table = jnp.pad(table, ((0, 0), (0, 128 - d0 % 128)))
    dd = table.shape[1]
    e = idx.shape[0]
    info = plsc.get_sparse_core_info()
    nc, ns = info.num_cores, info.num_subcores
    nw = nc * ns
    per_w = e // nw
    ch = 512
    nch = per_w // ch
    mesh = plsc.VectorSubcoreMesh(core_axis_name="c", subcore_axis_name="s")

    @functools.partial(
        pl.kernel, mesh=mesh,
        out_type=jax.ShapeDtypeStruct((e, dd), jnp.float32),
        scratch_types=[
            pltpu.VMEM((ch,), jnp.int32),
            pltpu.VMEM((ch, dd), jnp.float32),
            pltpu.SemaphoreType.DMA,
        ],
    )
    def gk(idx_hbm, table_hbm, out_hbm, idx_v, rows_v, sem):
        wid = jax.lax.axis_index("s") * nc + jax.lax.axis_index("c")
        base = wid * per_w

        def body(ci, _):
            off = base + ci * ch
            pltpu.sync_copy(idx_hbm.at[pl.ds(off, ch)], idx_v)
            pltpu.async_copy(table_hbm.at[idx_v], rows_v, sem).wait()
            pltpu.sync_copy(rows_v, out_hbm.at[pl.ds(off, ch)])
            return 0

        jax.lax.fori_loop(0, nch, body, 0)

    out = gk(idx, table)
    return out[:, :d0] if d0 != dd else out


def _gnorm(x, w, b, num_groups=8, eps=1e-5):
    sh = x.shape
    c = sh[-1]
    xg = x.reshape(sh[:-1] + (num_groups, c // num_groups))
    mean = jnp.mean(xg, -1, keepdims=True)
    var = jnp.var(xg, -1, keepdims=True)
    xg = (xg - mean) / jnp.sqrt(var + eps)
    return xg.reshape(sh) * w + b


def _set_conv(p, signal, pts_flat, edges):
    b, n, c = signal.shape
    k = _K
    cout = p['W1'].shape[0]
    w1a = p['W1'][:, :c]
    w1b = p['W1'][:, c:]
    sig = signal.reshape(b * n, c)
    pw = pts_flat @ w1b.T
    u = sig @ w1a.T + pw
    v = p['b1'][None, :] - pw
    h = _sc_gather(u, edges).reshape(b * n, k, cout) + v[:, None, :]
    h = _gnorm(h, p['gn1_w'], p['gn1_b'])
    h = jax.nn.leaky_relu(h, 0.1)
    h = jnp.max(h, axis=1)
    h = h @ p['W2'].T + p['b2']
    h = _gnorm(h, p['gn2_w'], p['gn2_b'])
    h = jax.nn.leaky_relu(h, 0.1)
    h = h @ p['W3'].T + p['b3']
    h = _gnorm(h, p['gn3_w'], p['gn3_b'])
    h = jax.nn.leaky_relu(h, 0.1)
    return h.reshape(b, n, cout)


def kernel(pc1, pc2, params):
    p1f = pc1.reshape(_B * _N, 3)
    p2f = pc2.reshape(_B * _M, 3)
    e1 = _build_graph(pc1, _K)
    f0 = _set_conv(params['feat_conv1'], pc1, p1f, e1)
    f0 = _set_conv(params['feat_conv2'], f0, p1f, e1)
    f0 = _set_conv(params['feat_conv3'], f0, p1f, e1)
    e2 = _build_graph(pc2, _K)
    f1 = _set_conv(params['feat_conv1'], pc2, p2f, e2)
    f1 = _set_conv(params['feat_conv2'], f1, p2f, e2)
    f1 = _set_conv(params['feat_conv3'], f1, p2f, e2)
    eps = jnp.exp(params['epsilon'])[0] + 0.03
    gam = jnp.exp(params['gamma'])[0]
    ot_flow = _sinkhorn_flow(f0, f1, pc1, pc2, eps, gam)
    x = _set_conv(params['ref_conv1'], ot_flow, p1f, e1)
    x = _set_conv(params['ref_conv2'], x, p1f, e1)
    x = _set_conv(params['ref_conv3'], x, p1f, e1)
    x = x.reshape(_B * _N, -1) @ params['fc_W'].T + params['fc_b']
    return ot_flow + x.reshape(_B, _N, 3)


# speedup vs baseline: 1.5053x; 1.0038x over previous
import functools

import jax
import jax.numpy as jnp
import numpy as np
from jax.experimental import pallas as pl
from jax.experimental.pallas import tpu as pltpu
from jax.experimental.pallas import tpu_sc as plsc

_N_HID = 32
_NB_ITER = 1
_K = 32
_B, _N, _M = 2, 4096, 4096


_BI = 256


def _ktile(f0b, f1, p1b, p2, rec_eps):
    f0n = f0b * jax.lax.rsqrt(jnp.sum(f0b * f0b, -1, keepdims=True) + 1e-8)
    f1n = f1 * jax.lax.rsqrt(jnp.sum(f1 * f1, -1, keepdims=True) + 1e-8)
    c = 1.0 - jax.lax.dot_general(
        f0n, f1n, (((1,), (1,)), ((), ())), preferred_element_type=jnp.float32)
    d2 = jnp.sum(p1b * p1b, -1, keepdims=True) + jnp.sum(p2 * p2, -1)[None, :]
    d2 = d2 - 2.0 * jax.lax.dot_general(
        p1b, p2, (((1,), (1,)), ((), ())), preferred_element_type=jnp.float32)
    return jnp.exp(-c * rec_eps) * (d2 < 100.0).astype(jnp.float32)


def _colsum_body(scal, f0, f1, p1, p2, out):
    i = pl.program_id(1)
    kt = _ktile(f0[0], f1[0], p1[0], p2[0], scal[0])
    part = jnp.sum(kt, axis=0, keepdims=True)

    @pl.when(i == 0)
    def _():
        out[0] = part

    @pl.when(i != 0)
    def _():
        out[0] += part


def _rows_body(scal, f0, f1, p1, p2, colsum, out):
    rec_eps, power, inv_n, inv_m = scal[0], scal[1], scal[2], scal[3]
    kt = _ktile(f0[0], f1[0], p1[0], p2[0], rec_eps)
    bb = jnp.exp(power * jnp.log(inv_m / (colsum[0, 0] * inv_n + 1e-8)))
    kb_t = kt * bb[None, :]
    kb = jnp.sum(kb_t, -1, keepdims=True)
    a = jnp.exp(power * jnp.log(inv_n / (kb + 1e-8)))
    s = jnp.dot(kb_t, p2[0], preferred_element_type=jnp.float32)
    out[0] = (a * s) / (a * kb + 1e-8) - p1[0]


def _sinkhorn_flow(f0, f1, pc1, pc2, eps, gam):
    power = gam / (gam + eps)
    scal = jnp.stack([1.0 / eps, power, 1.0 / _N, 1.0 / _M]).astype(jnp.float32)
    n_i = _N // _BI
    colsum = pl.pallas_call(
        _colsum_body,
        grid=(_B, n_i),
        in_specs=[
            pl.BlockSpec(memory_space=pltpu.SMEM),
            pl.BlockSpec((1, _BI, f0.shape[-1]), lambda b, i: (b, i, 0)),
            pl.BlockSpec((1, _M, f1.shape[-1]), lambda b, i: (b, 0, 0)),
            pl.BlockSpec((1, _BI, 3), lambda b, i: (b, i, 0)),
            pl.BlockSpec((1, _M, 3), lambda b, i: (b, 0, 0)),
        ],
        out_specs=pl.BlockSpec((1, 1, _M), lambda b, i: (b, 0, 0)),
        out_shape=jax.ShapeDtypeStruct((_B, 1, _M), jnp.float32),
    )(scal, f0, f1, pc1, pc2)
    flow = pl.pallas_call(
        _rows_body,
        grid=(_B, n_i),
        in_specs=[
            pl.BlockSpec(memory_space=pltpu.SMEM),
            pl.BlockSpec((1, _BI, f0.shape[-1]), lambda b, i: (b, i, 0)),
            pl.BlockSpec((1, _M, f1.shape[-1]), lambda b, i: (b, 0, 0)),
            pl.BlockSpec((1, _BI, 3), lambda b, i: (b, i, 0)),
            pl.BlockSpec((1, _M, 3), lambda b, i: (b, 0, 0)),
            pl.BlockSpec((1, 1, _M), lambda b, i: (b, 0, 0)),
        ],
        out_specs=pl.BlockSpec((1, _BI, 3), lambda b, i: (b, i, 0)),
        out_shape=jax.ShapeDtypeStruct((_B, _N, 3), jnp.float32),
    )(scal, f0, f1, pc1, pc2, colsum)
    return flow


_RB = 128
_SEG = 512


def _knn_body(h_rows, g_cols, out, d_ref, acc_ref, pidx_ref):
    t = pl.program_id(2)
    n = g_cols.shape[1]
    nseg = n // _SEG
    big = jnp.int32(2 ** 30)

    @pl.when(t == 0)
    def _():
        hr = h_rows[0]

        def init(s, _):
            ga = g_cols[0, pl.ds(s * _SEG, _SEG)]
            s3 = ga[:, 0:1] * hr[0:1, :]
            s3 = s3 + ga[:, 1:2] * hr[1:2, :]
            s3 = s3 + ga[:, 2:3] * hr[2:3, :]
            d_ref[pl.ds(s * _SEG, _SEG), :] = (
                jnp.broadcast_to(ga[:, 3:4], (_SEG, _RB)) + s3)
            return 0

        jax.lax.fori_loop(0, nseg, init, 0, unroll=2)
        pidx_ref[...] = jnp.full((1, _RB), -1, jnp.int32)

    pidx = pidx_ref[...]

    def scan(s, carry):
        m, idx = carry
        seg = d_ref[pl.ds(s * _SEG, _SEG), :]
        jidx = jax.lax.broadcasted_iota(jnp.int32, (_SEG, _RB), 0) + s * _SEG
        seg = jnp.where(jidx == pidx, jnp.float32(1e30), seg)
        d_ref[pl.ds(s * _SEG, _SEG), :] = seg
        smin = jnp.min(seg, 0, keepdims=True)
        scand = jnp.min(jnp.where(seg == smin, jidx, big), 0, keepdims=True)
        better = smin < m
        return (jnp.where(better, smin, m), jnp.where(better, scand, idx))

    m0 = jnp.full((1, _RB), 1e30, jnp.float32)
    i0 = jnp.full((1, _RB), big, jnp.int32)
    _, idx = jax.lax.fori_loop(0, nseg, scan, (m0, i0), unroll=2)
    pidx_ref[...] = idx
    acc_ref[pl.ds(t, 1)] = idx

    @pl.when(t == _K - 1)
    def _():
        out[0] = acc_ref[...]


def _build_graph(pcloud, k):
    b, n, _ = pcloud.shape
    p2sum = jnp.sum(pcloud * pcloud, -1, keepdims=True)
    pr = jax.lax.reduce_precision(pcloud, 8, 7)
    g = jnp.concatenate([pr, p2sum], -1)
    h = -2.0 * jnp.swapaxes(pr, 1, 2)
    nbr_t = pl.pallas_call(
        _knn_body,
        grid=(b, n // _RB, _K),
        in_specs=[
            pl.BlockSpec((1, 3, _RB), lambda bb, i, t: (bb, 0, i)),
            pl.BlockSpec((1, n, 4), lambda bb, i, t: (bb, 0, 0)),
        ],
        out_specs=pl.BlockSpec((1, _K, _RB), lambda bb, i, t: (bb, 0, i)),
        out_shape=jax.ShapeDtypeStruct((b, _K, n), jnp.int32),
        scratch_shapes=[pltpu.VMEM((n, _RB), jnp.float32),
                        pltpu.VMEM((_K, _RB), jnp.int32),
                        pltpu.VMEM((1, _RB), jnp.int32)],
    )(h, g)
    neighbors = jnp.swapaxes(nbr_t, 1, 2)
    edges = (neighbors + jnp.arange(b)[:, None, None] * n).reshape(-1)
    return edges


def _sc_gather(table, idx):
    v, d0 = table.shape
    if d0 % 128:
        table = jnp.pad(table, ((0, 0), (0, 128 - d0 % 128)))
    dd = table.shape[1]
    e = idx.shape[0]
    info = plsc.get_sparse_core_info()
    nc, ns = info.num_cores, info.num_subcores
    nw = nc * ns
    per_w = e // nw
    ch = 512
    nch = per_w // ch
    mesh = plsc.VectorSubcoreMesh(core_axis_name="c", subcore_axis_name="s")

    @functools.partial(
        pl.kernel, mesh=mesh,
        out_type=jax.ShapeDtypeStruct((e, dd), jnp.float32),
        scratch_types=[
            pltpu.VMEM((per_w,), jnp.int32),
            pltpu.VMEM((ch, dd), jnp.float32),
            pltpu.SemaphoreType.DMA,
        ],
    )
    def gk(idx_hbm, table_hbm, out_hbm, idx_v, rows_v, sem):
        wid = jax.lax.axis_index("s") * nc + jax.lax.axis_index("c")
        base = wid * per_w
        pltpu.sync_copy(idx_hbm.at[pl.ds(base, per_w)], idx_v)

        def body(ci, _):
            pltpu.async_copy(
                table_hbm.at[idx_v.at[pl.ds(ci * ch, ch)]], rows_v, sem).wait()
            pltpu.sync_copy(rows_v, out_hbm.at[pl.ds(base + ci * ch, ch)])
            return 0

        jax.lax.fori_loop(0, nch, body, 0)

    out = gk(idx, table)
    return out[:, :d0] if d0 != dd else out


def _gnorm(x, w, b, num_groups=8, eps=1e-5):
    sh = x.shape
    c = sh[-1]
    xg = x.reshape(sh[:-1] + (num_groups, c // num_groups))
    mean = jnp.mean(xg, -1, keepdims=True)
    var = jnp.var(xg, -1, keepdims=True)
    xg = (xg - mean) / jnp.sqrt(var + eps)
    return xg.reshape(sh) * w + b


def _set_conv(p, signal, pts_flat, edges):
    b, n, c = signal.shape
    k = _K
    cout = p['W1'].shape[0]
    w1a = p['W1'][:, :c]
    w1b = p['W1'][:, c:]
    sig = signal.reshape(b * n, c)
    pw = pts_flat @ w1b.T
    u = sig @ w1a.T + pw
    v = p['b1'][None, :] - pw
    h = _sc_gather(u, edges).reshape(b * n, k, cout) + v[:, None, :]
    h = _gnorm(h, p['gn1_w'], p['gn1_b'])
    h = jax.nn.leaky_relu(h, 0.1)
    h = jnp.max(h, axis=1)
    h = h @ p['W2'].T + p['b2']
    h = _gnorm(h, p['gn2_w'], p['gn2_b'])
    h = jax.nn.leaky_relu(h, 0.1)
    h = h @ p['W3'].T + p['b3']
    h = _gnorm(h, p['gn3_w'], p['gn3_b'])
    h = jax.nn.leaky_relu(h, 0.1)
    return h.reshape(b, n, cout)


def kernel(pc1, pc2, params):
    p1f = pc1.reshape(_B * _N, 3)
    p2f = pc2.reshape(_B * _M, 3)
    e1 = _build_graph(pc1, _K)
    f0 = _set_conv(params['feat_conv1'], pc1, p1f, e1)
    f0 = _set_conv(params['feat_conv2'], f0, p1f, e1)
    f0 = _set_conv(params['feat_conv3'], f0, p1f, e1)
    e2 = _build_graph(pc2, _K)
    f1 = _set_conv(params['feat_conv1'], pc2, p2f, e2)
    f1 = _set_conv(params['feat_conv2'], f1, p2f, e2)
    f1 = _set_conv(params['feat_conv3'], f1, p2f, e2)
    eps = jnp.exp(params['epsilon'])[0] + 0.03
    gam = jnp.exp(params['gamma'])[0]
    ot_flow = _sinkhorn_flow(f0, f1, pc1, pc2, eps, gam)
    x = _set_conv(params['ref_conv1'], ot_flow, p1f, e1)
    x = _set_conv(params['ref_conv2'], x, p1f, e1)
    x = _set_conv(params['ref_conv3'], x, p1f, e1)
    x = x.reshape(_B * _N, -1) @ params['fc_W'].T + params['fc_b']
    return ot_flow + x.reshape(_B, _N, 3)
